# baseline (speedup 1.0000x reference)
import functools

import jax
import jax.numpy as jnp
from jax import lax
from jax.experimental import pallas as pl
from jax.experimental.pallas import tpu as pltpu
from jax.experimental.pallas import tpu_sc as plsc

N = 10000
NP = 10240
D = 128
E = 320000
CH = 128
NC = 2
NS = 16
TILES = NC * NS
C0 = 106
C1 = 54
NCH = NS * (C0 + C1)
NCHP = NCH + (C0 - C1)
EP = NCHP * CH
DPT = NCH // TILES
RPS = NP // NS
RB = 1024
GRID = NP // RB

_f32 = jnp.float32
_mesh = plsc.VectorSubcoreMesh(
    core_axis_name="c", subcore_axis_name="s", num_cores=NC, num_subcores=NS)


@functools.partial(
    pl.kernel,
    out_type=jax.ShapeDtypeStruct((NC * NP,), _f32),
    mesh=_mesh,
    scratch_types=[
        pltpu.VMEM((DPT, 2, CH), jnp.int32),
        pltpu.VMEM((CH,), _f32),
        pltpu.VMEM((RPS,), _f32),
        pltpu.VMEM_SHARED((NP,), _f32),
    ],
)
def _sc_deg(edges_hbm, degp_hbm, idx_v, ones_v, zeros_v, deg_sh):
    cid = lax.axis_index("c")
    sid = lax.axis_index("s")
    wid = cid * NS + sid
    for i in range(RPS // 16):
        zeros_v[pl.ds(i * 16, 16)] = jnp.zeros((16,), _f32)
    for i in range(CH // 16):
        ones_v[pl.ds(i * 16, 16)] = jnp.full((16,), 1.0, _f32)
    pltpu.sync_copy(zeros_v, deg_sh.at[pl.ds(sid * RPS, RPS)])
    plsc.subcore_barrier()
    pltpu.sync_copy(edges_hbm.at[pl.ds(wid * DPT, DPT)], idx_v)

    def body(j, carry):
        pltpu.sync_copy(ones_v, deg_sh.at[idx_v.at[j, 1]], add=True)
        return carry

    lax.fori_loop(0, DPT, body, 0)
    plsc.subcore_barrier()
    pltpu.sync_copy(deg_sh.at[pl.ds(sid * RPS, RPS)],
                    degp_hbm.at[pl.ds(cid * NP + sid * RPS, RPS)])


@functools.partial(
    pl.kernel,
    out_type=jax.ShapeDtypeStruct((NC, NP, D), _f32),
    mesh=_mesh,
    scratch_types=[
        pltpu.VMEM((C0, 2, CH), jnp.int32),
        pltpu.VMEM((CH, D), _f32),
        pltpu.VMEM_SHARED((NP, D), _f32),
        pltpu.SemaphoreType.DMA,
    ],
)
def _sc_edge(xs_hbm, edges_hbm, accp_hbm, idxb, rows_v, acc_sh, gsem):
    cid = lax.axis_index("c")
    sid = lax.axis_index("s")
    base = jnp.where(cid == 0, sid * C0, NS * C0 + sid * C1)
    cnt = jnp.where(cid == 0, C0, C1)

    def zrow(i, carry):
        for k in range(D // 16):
            rows_v[i, pl.ds(k * 16, 16)] = jnp.zeros((16,), _f32)
        return carry

    lax.fori_loop(0, CH, zrow, 0)
    for t in range(RPS // CH):
        pltpu.sync_copy(rows_v, acc_sh.at[pl.ds(sid * RPS + t * CH, CH)])
    plsc.subcore_barrier()

    pltpu.sync_copy(edges_hbm.at[pl.ds(base, C0)], idxb)

    def body(j, carry):
        pltpu.async_copy(xs_hbm.at[idxb.at[j, 0]], rows_v, gsem).wait()
        pltpu.sync_copy(rows_v, acc_sh.at[idxb.at[j, 1]], add=True)
        return carry

    lax.fori_loop(0, cnt, body, 0)
    plsc.subcore_barrier()
    pltpu.sync_copy(acc_sh.at[pl.ds(sid * RPS, RPS)],
                    accp_hbm.at[cid, pl.ds(sid * RPS, RPS)])


def _tc_first_body(x_ref, w_ref, degp_ref, xs_ref, dinv_ref):
    deg = degp_ref[0] + degp_ref[1] + 1.0
    dinv = lax.rsqrt(deg)
    dinv_ref[...] = dinv
    xs_ref[...] = dinv * jnp.dot(x_ref[...], w_ref[...],
                                 preferred_element_type=_f32)


_tc_first = pl.pallas_call(
    _tc_first_body,
    grid=(GRID,),
    in_specs=[
        pl.BlockSpec((RB, D), lambda i: (i, 0)),
        pl.BlockSpec((D, D), lambda i: (0, 0)),
        pl.BlockSpec((NC, RB, 1), lambda i: (0, i, 0)),
    ],
    out_specs=[
        pl.BlockSpec((RB, D), lambda i: (i, 0)),
        pl.BlockSpec((RB, 1), lambda i: (i, 0)),
    ],
    out_shape=[
        jax.ShapeDtypeStruct((NP, D), _f32),
        jax.ShapeDtypeStruct((NP, 1), _f32),
    ],
)


def _tc_mid_body(acc_ref, xs_ref, dinv_ref, w_ref, b_ref, xs2_ref):
    a = acc_ref[0] + acc_ref[1] + xs_ref[...]
    h = jnp.maximum(dinv_ref[...] * a + b_ref[...], 0.0)
    xs2_ref[...] = dinv_ref[...] * jnp.dot(h, w_ref[...],
                                           preferred_element_type=_f32)


_tc_mid = pl.pallas_call(
    _tc_mid_body,
    grid=(GRID,),
    in_specs=[
        pl.BlockSpec((NC, RB, D), lambda i: (0, i, 0)),
        pl.BlockSpec((RB, D), lambda i: (i, 0)),
        pl.BlockSpec((RB, 1), lambda i: (i, 0)),
        pl.BlockSpec((D, D), lambda i: (0, 0)),
        pl.BlockSpec((1, D), lambda i: (0, 0)),
    ],
    out_specs=pl.BlockSpec((RB, D), lambda i: (i, 0)),
    out_shape=jax.ShapeDtypeStruct((NP, D), _f32),
)


def _tc_final_body(acc_ref, xs_ref, dinv_ref, b_ref, out_ref):
    a = acc_ref[0] + acc_ref[1] + xs_ref[...]
    out_ref[...] = dinv_ref[...] * a + b_ref[...]


_tc_final = pl.pallas_call(
    _tc_final_body,
    grid=(GRID,),
    in_specs=[
        pl.BlockSpec((NC, RB, D), lambda i: (0, i, 0)),
        pl.BlockSpec((RB, D), lambda i: (i, 0)),
        pl.BlockSpec((RB, 1), lambda i: (i, 0)),
        pl.BlockSpec((1, D), lambda i: (0, 0)),
    ],
    out_specs=pl.BlockSpec((RB, D), lambda i: (i, 0)),
    out_shape=jax.ShapeDtypeStruct((NP, D), _f32),
)


def kernel(x, edge_index, W1, b1, W2, b2):
    src = edge_index[0]
    dst = edge_index[1]
    pad = jnp.full((EP - E,), N, jnp.int32)
    src_r = jnp.concatenate([src, pad]).reshape(NCHP, CH)
    dst_r = jnp.concatenate([dst, pad]).reshape(NCHP, CH)
    edges = jnp.stack([src_r, dst_r], axis=1)
    x_pad = jnp.pad(x, ((0, NP - N), (0, 0)))

    deg_p = _sc_deg(edges).reshape(NC, NP, 1)
    xs1, dinv = _tc_first(x_pad, W1, deg_p)
    acc1 = _sc_edge(xs1, edges)
    xs2 = _tc_mid(acc1, xs1, dinv, W2, b1.reshape(1, D))
    acc2 = _sc_edge(xs2, edges)
    out = _tc_final(acc2, xs2, dinv, b2.reshape(1, D))
    return out[:N]


# speedup vs baseline: 1.4230x; 1.4230x over previous
import functools

import jax
import jax.numpy as jnp
from jax import lax
from jax.experimental import pallas as pl
from jax.experimental.pallas import tpu as pltpu
from jax.experimental.pallas import tpu_sc as plsc

N = 10000
NP = 10240
D = 128
E = 320000
CH = 128
NC = 2
NS = 16
TILES = NC * NS
EP = ((E + TILES * CH - 1) // (TILES * CH)) * (TILES * CH)
CPT = EP // (TILES * CH)
RPS = NP // NS
RB = 1024
GRID = NP // RB

_f32 = jnp.float32
_mesh = plsc.VectorSubcoreMesh(
    core_axis_name="c", subcore_axis_name="s", num_cores=NC, num_subcores=NS)


@functools.partial(
    pl.kernel,
    out_type=jax.ShapeDtypeStruct((NC * NP,), _f32),
    mesh=_mesh,
    scratch_types=[
        pltpu.VMEM((CPT, CH), jnp.int32),
        pltpu.VMEM((CH,), _f32),
        pltpu.VMEM((RPS,), _f32),
        pltpu.VMEM_SHARED((NP,), _f32),
    ],
)
def _sc_deg(dst_hbm, degp_hbm, idx_v, ones_v, zeros_v, deg_sh):
    cid = lax.axis_index("c")
    sid = lax.axis_index("s")
    wid = cid * NS + sid
    for i in range(RPS // 16):
        zeros_v[pl.ds(i * 16, 16)] = jnp.zeros((16,), _f32)
    for i in range(CH // 16):
        ones_v[pl.ds(i * 16, 16)] = jnp.full((16,), 1.0, _f32)
    pltpu.sync_copy(zeros_v, deg_sh.at[pl.ds(sid * RPS, RPS)])
    plsc.subcore_barrier()
    pltpu.sync_copy(dst_hbm.at[wid], idx_v)

    def body(j, carry):
        pltpu.sync_copy(ones_v, deg_sh.at[idx_v.at[j]], add=True)
        return carry

    lax.fori_loop(0, CPT, body, 0)
    plsc.subcore_barrier()
    pltpu.sync_copy(deg_sh.at[pl.ds(sid * RPS, RPS)],
                    degp_hbm.at[pl.ds(cid * NP + sid * RPS, RPS)])


@functools.partial(
    pl.kernel,
    out_type=jax.ShapeDtypeStruct((NC, NP, D), _f32),
    mesh=_mesh,
    scratch_types=[
        pltpu.VMEM((CPT, CH), jnp.int32),
        pltpu.VMEM((CPT, CH), jnp.int32),
        pltpu.VMEM((CH, D), _f32),
        pltpu.VMEM_SHARED((NP, D), _f32),
        pltpu.SemaphoreType.DMA,
    ],
)
def _sc_edge(xs_hbm, src_hbm, dstr_hbm, accp_hbm, idx_s, idx_d, rows_v,
             acc_sh, sem):
    cid = lax.axis_index("c")
    sid = lax.axis_index("s")
    wid = cid * NS + sid

    def zrow(i, carry):
        for k in range(D // 16):
            rows_v[i, pl.ds(k * 16, 16)] = jnp.zeros((16,), _f32)
        return carry

    lax.fori_loop(0, CH, zrow, 0)
    for t in range(RPS // CH):
        pltpu.sync_copy(rows_v, acc_sh.at[pl.ds(sid * RPS + t * CH, CH)])
    plsc.subcore_barrier()

    pltpu.sync_copy(src_hbm.at[wid], idx_s)
    pltpu.sync_copy(dstr_hbm.at[wid], idx_d)

    def body(j, carry):
        pltpu.async_copy(xs_hbm.at[idx_s.at[j]], rows_v, sem).wait()
        pltpu.sync_copy(rows_v, acc_sh.at[idx_d.at[j]], add=True)
        return carry

    lax.fori_loop(0, CPT, body, 0)
    plsc.subcore_barrier()
    pltpu.sync_copy(acc_sh.at[pl.ds(sid * RPS, RPS)],
                    accp_hbm.at[cid, pl.ds(sid * RPS, RPS)])


def _tc_first_body(x_ref, w_ref, degp_ref, xs_ref, dinv_ref):
    deg = degp_ref[0] + degp_ref[1] + 1.0
    dinv = lax.rsqrt(deg)
    dinv_ref[...] = dinv
    xs_ref[...] = dinv * jnp.dot(x_ref[...], w_ref[...],
                                 preferred_element_type=_f32)


_tc_first = pl.pallas_call(
    _tc_first_body,
    grid=(GRID,),
    in_specs=[
        pl.BlockSpec((RB, D), lambda i: (i, 0)),
        pl.BlockSpec((D, D), lambda i: (0, 0)),
        pl.BlockSpec((NC, RB, 1), lambda i: (0, i, 0)),
    ],
    out_specs=[
        pl.BlockSpec((RB, D), lambda i: (i, 0)),
        pl.BlockSpec((RB, 1), lambda i: (i, 0)),
    ],
    out_shape=[
        jax.ShapeDtypeStruct((NP, D), _f32),
        jax.ShapeDtypeStruct((NP, 1), _f32),
    ],
)


def _tc_mid_body(acc_ref, xs_ref, dinv_ref, w_ref, b_ref, xs2_ref):
    a = acc_ref[0] + acc_ref[1] + xs_ref[...]
    h = jnp.maximum(dinv_ref[...] * a + b_ref[...], 0.0)
    xs2_ref[...] = dinv_ref[...] * jnp.dot(h, w_ref[...],
                                           preferred_element_type=_f32)


_tc_mid = pl.pallas_call(
    _tc_mid_body,
    grid=(GRID,),
    in_specs=[
        pl.BlockSpec((NC, RB, D), lambda i: (0, i, 0)),
        pl.BlockSpec((RB, D), lambda i: (i, 0)),
        pl.BlockSpec((RB, 1), lambda i: (i, 0)),
        pl.BlockSpec((D, D), lambda i: (0, 0)),
        pl.BlockSpec((1, D), lambda i: (0, 0)),
    ],
    out_specs=pl.BlockSpec((RB, D), lambda i: (i, 0)),
    out_shape=jax.ShapeDtypeStruct((NP, D), _f32),
)


def _tc_final_body(acc_ref, xs_ref, dinv_ref, b_ref, out_ref):
    a = acc_ref[0] + acc_ref[1] + xs_ref[...]
    out_ref[...] = dinv_ref[...] * a + b_ref[...]


_tc_final = pl.pallas_call(
    _tc_final_body,
    grid=(GRID,),
    in_specs=[
        pl.BlockSpec((NC, RB, D), lambda i: (0, i, 0)),
        pl.BlockSpec((RB, D), lambda i: (i, 0)),
        pl.BlockSpec((RB, 1), lambda i: (i, 0)),
        pl.BlockSpec((1, D), lambda i: (0, 0)),
    ],
    out_specs=pl.BlockSpec((RB, D), lambda i: (i, 0)),
    out_shape=jax.ShapeDtypeStruct((NP, D), _f32),
)


def kernel(x, edge_index, W1, b1, W2, b2):
    src = edge_index[0]
    dst = edge_index[1]
    pad = jnp.full((EP - E,), N, jnp.int32)
    src_r = jnp.concatenate([src, pad]).reshape(TILES, CPT, CH)
    dst_r = jnp.concatenate([dst, pad]).reshape(TILES, CPT, CH)
    x_pad = jnp.pad(x, ((0, NP - N), (0, 0)))

    deg_p = _sc_deg(dst_r).reshape(NC, NP, 1)
    xs1, dinv = _tc_first(x_pad, W1, deg_p)
    acc1 = _sc_edge(xs1, src_r, dst_r)
    xs2 = _tc_mid(acc1, xs1, dinv, W2, b1.reshape(1, D))
    acc2 = _sc_edge(xs2, src_r, dst_r)
    out = _tc_final(acc2, xs2, dinv, b2.reshape(1, D))
    return out[:N]
